# baseline (speedup 1.0000x reference)
import functools

import jax
import jax.numpy as jnp
from jax import lax
from jax.experimental import pallas as pl
from jax.experimental.pallas import tpu as pltpu
from jax.experimental.pallas import tpu_sc as plsc

B, D = 16384, 128
NC, NS = 2, 16
NW = NC * NS
ROWS_W = B // NW
CH = 128
RANK = D // 2
LANES = 16
G = RANK // LANES

_mesh = plsc.VectorSubcoreMesh(core_axis_name="c", subcore_axis_name="s")


@functools.partial(
    pl.kernel,
    out_type=jax.ShapeDtypeStruct((B, D), jnp.float32),
    mesh=_mesh,
    scratch_types=[
        pltpu.VMEM((CH, D), jnp.float32),
        pltpu.VMEM((CH, D), jnp.float32),
        pltpu.VMEM((CH, D), jnp.float32),
    ],
)
def _sc_complex(lhs_hbm, rel_hbm, out_hbm, lhs_v, rel_v, out_v):
    wid = lax.axis_index("s") * NC + lax.axis_index("c")
    base = wid * ROWS_W

    def chunk(ci, carry):
        row0 = base + ci * CH
        pltpu.sync_copy(lhs_hbm.at[pl.ds(row0, CH)], lhs_v)
        pltpu.sync_copy(rel_hbm.at[pl.ds(row0, CH)], rel_v)

        def rowbody(i, c2):
            for g in range(G):
                lo = g * LANES
                hi = RANK + g * LANES
                l0 = lhs_v[i, pl.ds(lo, LANES)]
                l1 = lhs_v[i, pl.ds(hi, LANES)]
                r0 = rel_v[i, pl.ds(lo, LANES)]
                r1 = rel_v[i, pl.ds(hi, LANES)]
                out_v[i, pl.ds(lo, LANES)] = l0 * r0 - l1 * r1
                out_v[i, pl.ds(hi, LANES)] = l0 * r1 + l1 * r0
            return c2

        lax.fori_loop(0, CH, rowbody, 0)
        pltpu.sync_copy(out_v, out_hbm.at[pl.ds(row0, CH)])
        return carry

    lax.fori_loop(0, ROWS_W // CH, chunk, 0)


def kernel(lhs, rel):
    return _sc_complex(lhs, rel)


# speedup vs baseline: 1.2058x; 1.2058x over previous
import functools

import jax
import jax.numpy as jnp
from jax import lax
from jax.experimental import pallas as pl
from jax.experimental.pallas import tpu as pltpu
from jax.experimental.pallas import tpu_sc as plsc

B, D = 16384, 128
NC, NS = 2, 16
NW = NC * NS
ROWS_W = B // NW
CH = 128
NCH = ROWS_W // CH
RANK = D // 2
LANES = 16
G = RANK // LANES

_mesh = plsc.VectorSubcoreMesh(core_axis_name="c", subcore_axis_name="s")


@functools.partial(
    pl.kernel,
    out_type=jax.ShapeDtypeStruct((B, D), jnp.float32),
    mesh=_mesh,
    scratch_types=[
        pltpu.VMEM((CH, D), jnp.float32),
        pltpu.VMEM((CH, D), jnp.float32),
        pltpu.VMEM((CH, D), jnp.float32),
        pltpu.VMEM((CH, D), jnp.float32),
        pltpu.VMEM((CH, D), jnp.float32),
        pltpu.VMEM((CH, D), jnp.float32),
        pltpu.SemaphoreType.DMA,
        pltpu.SemaphoreType.DMA,
        pltpu.SemaphoreType.DMA,
        pltpu.SemaphoreType.DMA,
        pltpu.SemaphoreType.DMA,
        pltpu.SemaphoreType.DMA,
    ],
)
def _sc_complex(lhs_hbm, rel_hbm, out_hbm, lv0, lv1, rv0, rv1, ov0, ov1,
                sl0, sl1, sr0, sr1, so0, so1):
    lv, rv, ov = [lv0, lv1], [rv0, rv1], [ov0, ov1]
    sl, sr, so = [sl0, sl1], [sr0, sr1], [so0, so1]

    wid = lax.axis_index("s") * NC + lax.axis_index("c")
    base = wid * ROWS_W

    def start_in(ci):
        b = ci % 2
        row0 = base + ci * CH
        cl = pltpu.make_async_copy(lhs_hbm.at[pl.ds(row0, CH)], lv[b], sl[b])
        cr = pltpu.make_async_copy(rel_hbm.at[pl.ds(row0, CH)], rv[b], sr[b])
        cl.start()
        cr.start()
        return cl, cr

    def start_out(ci):
        b = ci % 2
        row0 = base + ci * CH
        co = pltpu.make_async_copy(ov[b], out_hbm.at[pl.ds(row0, CH)], so[b])
        co.start()
        return co

    def compute(lhs_v, rel_v, out_v):
        def rowbody(i, c2):
            for g in range(G):
                lo = g * LANES
                hi = RANK + g * LANES
                l0 = lhs_v[i, pl.ds(lo, LANES)]
                l1 = lhs_v[i, pl.ds(hi, LANES)]
                r0 = rel_v[i, pl.ds(lo, LANES)]
                r1 = rel_v[i, pl.ds(hi, LANES)]
                out_v[i, pl.ds(lo, LANES)] = l0 * r0 - l1 * r1
                out_v[i, pl.ds(hi, LANES)] = l0 * r1 + l1 * r0
            return c2

        lax.fori_loop(0, CH, rowbody, 0)

    pend_in = {0: start_in(0)}
    pend_out = {}
    for ci in range(NCH):
        if ci + 1 < NCH:
            pend_in[ci + 1] = start_in(ci + 1)
        cl, cr = pend_in.pop(ci)
        cl.wait()
        cr.wait()
        if ci - 2 in pend_out:
            pend_out.pop(ci - 2).wait()
        compute(lv[ci % 2], rv[ci % 2], ov[ci % 2])
        pend_out[ci] = start_out(ci)
    for co in pend_out.values():
        co.wait()


def kernel(lhs, rel):
    return _sc_complex(lhs, rel)
